# baseline (speedup 1.0000x reference)
import functools

import jax
import jax.numpy as jnp
from jax import lax
from jax.experimental import pallas as pl
from jax.experimental.pallas import tpu as pltpu
from jax.experimental.pallas import tpu_sc as plsc

_NC = 2
_NS = 16
_CHUNK = 128
_NBUF = 4


@functools.partial(jax.jit, static_argnums=(4, 5))
def _spmm_sc(edges, attn, xp, bias, n_nodes, dim):
    e = attn.shape[0]
    assert e % (_NS * 16) == 0
    epw = e // _NS
    n_full = epw // _CHUNK
    tail_e = epw - n_full * _CHUNK
    n_main = n_full - n_full % _NBUF
    leftovers = [(ci, _CHUNK) for ci in range(n_main, n_full)]
    if tail_e:
        leftovers.append((n_full, tail_e))
    dc = dim // _NC
    assert n_main >= 2 * _NBUF
    rows_per_sub = (n_nodes // _NS) // 8 * 8
    tail_rows = n_nodes - _NS * rows_per_sub
    tail_r0 = _NS * rows_per_sub

    mesh = plsc.VectorSubcoreMesh(core_axis_name="c", subcore_axis_name="s")

    @functools.partial(
        pl.kernel,
        out_type=jax.ShapeDtypeStruct((n_nodes, dim), jnp.float32),
        mesh=mesh,
        scratch_types=[
            pltpu.VMEM_SHARED((n_nodes, dc), jnp.bfloat16),
            pltpu.VMEM_SHARED((n_nodes, dc), jnp.float32),
            pltpu.VMEM((_NBUF, 2, _CHUNK), jnp.int32),
            pltpu.VMEM((_NBUF, _CHUNK), jnp.float32),
            pltpu.VMEM((_NBUF, _CHUNK, dc), jnp.bfloat16),
            pltpu.VMEM((_NBUF, _CHUNK, dc), jnp.float32),
            pltpu.VMEM((2, _CHUNK), jnp.int32),
            pltpu.VMEM((_CHUNK,), jnp.float32),
        ] + [pltpu.SemaphoreType.DMA] * (3 * _NBUF),
        compiler_params=pltpu.CompilerParams(
            use_tc_tiling_on_sc=False, needs_layout_passes=False),
    )
    def k(edges_hbm, attn_hbm, x_hbm, binit_hbm, out_hbm,
          x_sh, acc_sh, ei_v, attn_v, rows_b, rows_f, ei_t, attn_t, *sems):
        sem_e = sems[:_NBUF]
        sem_g = sems[_NBUF:2 * _NBUF]
        sem_s = sems[2 * _NBUF:]
        c = lax.axis_index("c")
        s = lax.axis_index("s")
        base_w = s * epw
        col0 = pl.multiple_of(c * dc, 8)

        def stage(r0, nr):
            pltpu.sync_copy(x_hbm.at[pl.ds(r0, nr), pl.ds(col0, dc)],
                            x_sh.at[pl.ds(r0, nr)])
            pltpu.sync_copy(binit_hbm.at[pl.ds(r0, nr)],
                            acc_sh.at[pl.ds(r0, nr)])

        r0 = s * rows_per_sub
        stage(r0, rows_per_sub)
        if tail_rows:
            @pl.when(s == _NS - 1)
            def _():
                stage(tail_r0, tail_rows)
        plsc.subcore_barrier()

        def issue_idx(ci, b):
            base = base_w + ci * _CHUNK
            pltpu.async_copy(edges_hbm.at[0, pl.ds(base, _CHUNK)],
                             ei_v.at[b, 0], sem_e[b])
            pltpu.async_copy(edges_hbm.at[1, pl.ds(base, _CHUNK)],
                             ei_v.at[b, 1], sem_e[b])
            pltpu.async_copy(attn_hbm.at[pl.ds(base, _CHUNK)],
                             attn_v.at[b], sem_e[b])

        def wait_idx(b):
            pltpu.make_async_copy(edges_hbm.at[0, pl.ds(0, _CHUNK)],
                                  ei_v.at[b, 0], sem_e[b]).wait()
            pltpu.make_async_copy(edges_hbm.at[1, pl.ds(0, _CHUNK)],
                                  ei_v.at[b, 1], sem_e[b]).wait()
            pltpu.make_async_copy(attn_hbm.at[pl.ds(0, _CHUNK)],
                                  attn_v.at[b], sem_e[b]).wait()

        def issue_gather(b):
            pltpu.async_copy(x_sh.at[ei_v.at[b, 1]], rows_b.at[b], sem_g[b])

        def wait_gather(b):
            pltpu.make_async_copy(x_sh.at[ei_v.at[b, 1]],
                                  rows_b.at[b], sem_g[b]).wait()

        def issue_scatter(b):
            pltpu.async_copy(rows_f.at[b], acc_sh.at[ei_v.at[b, 0]],
                             sem_s[b], add=True)

        def wait_scatter(b):
            pltpu.make_async_copy(rows_f.at[b], acc_sh.at[ei_v.at[b, 0]],
                                  sem_s[b]).wait()

        dnums = lax.GatherDimensionNumbers(
            offset_dims=(), collapsed_slice_dims=(0,), start_index_map=(0,))

        def scale_16rows(rb, rf, av, g):
            for r in range(16):
                row = g * 16 + r
                lane_idx = (jnp.zeros((16,), jnp.int32) + r)[:, None]
                a = lax.gather(av, lane_idx, dnums, (1,),
                               mode=lax.GatherScatterMode.PROMISE_IN_BOUNDS)
                for h in range(dc // 32):
                    packed = rb[row, pl.ds(h * 32, 32)]
                    lo, hi = plsc.unpack(
                        packed, format=plsc.PackFormat.INTERLEAVED)
                    rf[row, pl.ds(h * 32, 16)] = lo * a
                    rf[row, pl.ds(h * 32 + 16, 16)] = hi * a

        def scale(b):
            rb = rows_b.at[b]
            rf = rows_f.at[b]

            def scale_group(g, carry2):
                scale_16rows(rb, rf, attn_v[b, pl.ds(g * 16, 16)], g)
                return carry2

            lax.fori_loop(0, _CHUNK // 16, scale_group, 0)

        issue_idx(0, 0)
        issue_idx(1, 1)
        wait_idx(0)
        issue_gather(0)

        def quad_body(i4, carry):
            for kk in range(_NBUF):
                ci = i4 * _NBUF + kk
                b, b1, b2 = kk, (kk + 1) % _NBUF, (kk + 2) % _NBUF

                @pl.when(ci >= 2)
                def _():
                    wait_scatter(b2)

                @pl.when(ci + 2 < n_main)
                def _():
                    issue_idx(ci + 2, b2)

                @pl.when(ci + 1 < n_main)
                def _():
                    wait_idx(b1)
                    issue_gather(b1)

                wait_gather(b)
                scale(b)
                issue_scatter(b)
            return carry

        lax.fori_loop(0, n_main // _NBUF, quad_body, 0)
        wait_scatter((n_main - 2) % _NBUF)
        wait_scatter((n_main - 1) % _NBUF)

        for ci, cnt in leftovers:
            base = base_w + ci * _CHUNK
            pltpu.sync_copy(edges_hbm.at[:, pl.ds(base, cnt)],
                            ei_t.at[:, pl.ds(0, cnt)])
            pltpu.sync_copy(attn_hbm.at[pl.ds(base, cnt)],
                            attn_t.at[pl.ds(0, cnt)])
            pltpu.async_copy(x_sh.at[ei_t.at[1, pl.ds(0, cnt)]],
                             rows_b.at[0, pl.ds(0, cnt)], sem_g[0]).wait()
            for g in range(cnt // 16):
                scale_16rows(rows_b.at[0], rows_f.at[0],
                             attn_t[pl.ds(g * 16, 16)], g)
            pltpu.async_copy(rows_f.at[0, pl.ds(0, cnt)],
                             acc_sh.at[ei_t.at[0, pl.ds(0, cnt)]],
                             sem_s[0], add=True).wait()

        plsc.subcore_barrier()
        pltpu.sync_copy(acc_sh.at[pl.ds(r0, rows_per_sub)],
                        out_hbm.at[pl.ds(r0, rows_per_sub), pl.ds(col0, dc)])
        if tail_rows:
            @pl.when(s == _NS - 1)
            def _():
                pltpu.sync_copy(
                    acc_sh.at[pl.ds(tail_r0, tail_rows)],
                    out_hbm.at[pl.ds(tail_r0, tail_rows), pl.ds(col0, dc)])

    binit = jnp.zeros((n_nodes, dc), jnp.float32) + bias
    return k(edges, attn, xp, binit)


def kernel(edges, attentions, N, X):
    n_nodes, dim = X.shape
    e = attentions.shape[0]
    edges = edges.astype(jnp.int32)
    attentions = attentions.astype(jnp.float32)
    quantum = _NS * 16
    if e % quantum:
        pad = quantum - e % quantum
        edges = jnp.concatenate(
            [edges, jnp.zeros((2, pad), jnp.int32)], axis=1)
        attentions = jnp.concatenate(
            [attentions, jnp.zeros((pad,), jnp.float32)])
    xp = (X.reshape(n_nodes, dim // 32, 2, 16)
          .transpose(0, 1, 3, 2)
          .reshape(n_nodes, dim)
          .astype(jnp.bfloat16))
    bias = jnp.asarray(N, jnp.float32) - jnp.float32(n_nodes)
    return _spmm_sc(edges, attentions, xp, bias, n_nodes, dim)


# speedup vs baseline: 1.0749x; 1.0749x over previous
import functools

import jax
import jax.numpy as jnp
from jax import lax
from jax.experimental import pallas as pl
from jax.experimental.pallas import tpu as pltpu
from jax.experimental.pallas import tpu_sc as plsc

_NC = 2
_NS = 16
_CHUNK = 128
_NBUF = 4


@functools.partial(jax.jit, static_argnums=(4, 5))
def _spmm_sc(edges, attn, x, bias, n_nodes, dim):
    e = attn.shape[0]
    assert e % (_NS * 16) == 0
    epw = e // _NS
    n_full = epw // _CHUNK
    tail_e = epw - n_full * _CHUNK
    n_main = n_full - n_full % _NBUF
    leftovers = [(ci, _CHUNK) for ci in range(n_main, n_full)]
    if tail_e:
        leftovers.append((n_full, tail_e))
    dc = dim // _NC
    assert n_main >= 2 * _NBUF
    rows_per_sub = (n_nodes // _NS) // 8 * 8
    tail_rows = n_nodes - _NS * rows_per_sub
    tail_r0 = _NS * rows_per_sub

    mesh = plsc.VectorSubcoreMesh(core_axis_name="c", subcore_axis_name="s")

    @functools.partial(
        pl.kernel,
        out_type=jax.ShapeDtypeStruct((n_nodes, dim), jnp.float32),
        mesh=mesh,
        scratch_types=[
            pltpu.VMEM_SHARED((2, n_nodes, dc), jnp.float32),
            pltpu.VMEM((_NBUF, 2, _CHUNK), jnp.int32),
            pltpu.VMEM((_NBUF, _CHUNK), jnp.float32),
            pltpu.VMEM((_NBUF, _CHUNK, dc), jnp.float32),
            pltpu.VMEM((2, _CHUNK), jnp.int32),
            pltpu.VMEM((_CHUNK,), jnp.float32),
        ] + [pltpu.SemaphoreType.DMA] * (3 * _NBUF),
        compiler_params=pltpu.CompilerParams(
            use_tc_tiling_on_sc=False, needs_layout_passes=False),
    )
    def k(edges_hbm, attn_hbm, x_hbm, binit_hbm, out_hbm,
          sh, ei_v, attn_v, rows_v, ei_t, attn_t, *sems):
        x_sh = sh.at[0]
        acc_sh = sh.at[1]
        sem_e = sems[:_NBUF]
        sem_g = sems[_NBUF:2 * _NBUF]
        sem_s = sems[2 * _NBUF:]
        c = lax.axis_index("c")
        s = lax.axis_index("s")
        base_w = s * epw
        col0 = pl.multiple_of(c * dc, 8)

        def stage(r0, nr):
            pltpu.sync_copy(x_hbm.at[pl.ds(r0, nr), pl.ds(col0, dc)],
                            x_sh.at[pl.ds(r0, nr)])
            pltpu.sync_copy(binit_hbm.at[pl.ds(r0, nr)],
                            acc_sh.at[pl.ds(r0, nr)])

        r0 = s * rows_per_sub
        stage(r0, rows_per_sub)
        if tail_rows:
            @pl.when(s == _NS - 1)
            def _():
                stage(tail_r0, tail_rows)
        plsc.subcore_barrier()

        def issue_idx(ci, b):
            base = base_w + ci * _CHUNK
            pltpu.async_copy(edges_hbm.at[0, pl.ds(base, _CHUNK)],
                             ei_v.at[b, 0], sem_e[b])
            pltpu.async_copy(edges_hbm.at[1, pl.ds(base, _CHUNK)],
                             ei_v.at[b, 1], sem_e[b])
            pltpu.async_copy(attn_hbm.at[pl.ds(base, _CHUNK)],
                             attn_v.at[b], sem_e[b])

        def wait_idx(b):
            pltpu.make_async_copy(edges_hbm.at[0, pl.ds(0, _CHUNK)],
                                  ei_v.at[b, 0], sem_e[b]).wait()
            pltpu.make_async_copy(edges_hbm.at[1, pl.ds(0, _CHUNK)],
                                  ei_v.at[b, 1], sem_e[b]).wait()
            pltpu.make_async_copy(attn_hbm.at[pl.ds(0, _CHUNK)],
                                  attn_v.at[b], sem_e[b]).wait()

        def issue_gather(b):
            pltpu.async_copy(x_sh.at[ei_v.at[b, 1]], rows_v.at[b], sem_g[b])

        def wait_gather(b):
            pltpu.make_async_copy(x_sh.at[ei_v.at[b, 1]],
                                  rows_v.at[b], sem_g[b]).wait()

        def issue_scatter(b):
            pltpu.async_copy(rows_v.at[b], acc_sh.at[ei_v.at[b, 0]],
                             sem_s[b], add=True)

        def wait_scatter(b):
            pltpu.make_async_copy(rows_v.at[b], acc_sh.at[ei_v.at[b, 0]],
                                  sem_s[b]).wait()

        dnums = lax.GatherDimensionNumbers(
            offset_dims=(), collapsed_slice_dims=(0,), start_index_map=(0,))

        def scale_16rows(rv, av, g):
            for r in range(16):
                row = g * 16 + r
                lane_idx = (jnp.zeros((16,), jnp.int32) + r)[:, None]
                a = lax.gather(av, lane_idx, dnums, (1,),
                               mode=lax.GatherScatterMode.PROMISE_IN_BOUNDS)
                for cb in range(dc // 16):
                    sl = pl.ds(cb * 16, 16)
                    rv[row, sl] = rv[row, sl] * a

        def scale(b):
            rv = rows_v.at[b]

            def scale_group(g, carry2):
                scale_16rows(rv, attn_v[b, pl.ds(g * 16, 16)], g)
                return carry2

            lax.fori_loop(0, _CHUNK // 16, scale_group, 0)

        issue_idx(0, 0)
        issue_idx(1, 1)
        wait_idx(0)
        issue_gather(0)

        def quad_body(i4, carry):
            for kk in range(_NBUF):
                ci = i4 * _NBUF + kk
                b, b1, b2 = kk, (kk + 1) % _NBUF, (kk + 2) % _NBUF

                @pl.when(ci >= 2)
                def _():
                    wait_scatter(b2)

                @pl.when(ci + 2 < n_main)
                def _():
                    issue_idx(ci + 2, b2)

                @pl.when(ci + 1 < n_main)
                def _():
                    wait_idx(b1)
                    issue_gather(b1)

                wait_gather(b)
                scale(b)
                issue_scatter(b)
            return carry

        lax.fori_loop(0, n_main // _NBUF, quad_body, 0)
        wait_scatter((n_main - 2) % _NBUF)
        wait_scatter((n_main - 1) % _NBUF)

        for ci, cnt in leftovers:
            base = base_w + ci * _CHUNK
            pltpu.sync_copy(edges_hbm.at[:, pl.ds(base, cnt)], ei_t.at[:, pl.ds(0, cnt)])
            pltpu.sync_copy(attn_hbm.at[pl.ds(base, cnt)], attn_t.at[pl.ds(0, cnt)])
            rt = rows_v.at[0, pl.ds(0, cnt)]
            pltpu.async_copy(x_sh.at[ei_t.at[1, pl.ds(0, cnt)]], rt,
                             sem_g[0]).wait()
            rv = rows_v.at[0]
            for g in range(cnt // 16):
                scale_16rows(rv, attn_t[pl.ds(g * 16, 16)], g)
            pltpu.async_copy(rt, acc_sh.at[ei_t.at[0, pl.ds(0, cnt)]],
                             sem_s[0], add=True).wait()

        plsc.subcore_barrier()
        pltpu.sync_copy(acc_sh.at[pl.ds(r0, rows_per_sub)],
                        out_hbm.at[pl.ds(r0, rows_per_sub), pl.ds(col0, dc)])
        if tail_rows:
            @pl.when(s == _NS - 1)
            def _():
                pltpu.sync_copy(
                    acc_sh.at[pl.ds(tail_r0, tail_rows)],
                    out_hbm.at[pl.ds(tail_r0, tail_rows), pl.ds(col0, dc)])

    binit = jnp.zeros((n_nodes, dc), jnp.float32) + bias
    return k(edges, attn, x, binit)


def kernel(edges, attentions, N, X):
    n_nodes, dim = X.shape
    e = attentions.shape[0]
    edges = edges.astype(jnp.int32)
    attentions = attentions.astype(jnp.float32)
    quantum = _NS * 16
    if e % quantum:
        pad = quantum - e % quantum
        edges = jnp.concatenate(
            [edges, jnp.zeros((2, pad), jnp.int32)], axis=1)
        attentions = jnp.concatenate(
            [attentions, jnp.zeros((pad,), jnp.float32)])
    bias = jnp.asarray(N, jnp.float32) - jnp.float32(n_nodes)
    return _spmm_sc(edges, attentions, X, bias, n_nodes, dim)
